# baseline (speedup 1.0000x reference)
import functools

import jax
import jax.numpy as jnp
from jax import lax
from jax.experimental import pallas as pl
from jax.experimental.pallas import tpu as pltpu
from jax.experimental.pallas import tpu_sc as plsc

B = 16384
D = 64
NC = 2
NS = 16
NW = NC * NS
BPW = B // NW
C = 256


def _tec_body(head_hbm, rel_hbm, tail_hbm, ent_hbm, relt_hbm, nrm_hbm,
              out_hbm, hidx, tidx, ridx, hrows, trows, rrows, wrows, oscr,
              sem):
    wid = lax.axis_index("s") * NC + lax.axis_index("c")
    base = wid * BPW

    pltpu.sync_copy(head_hbm.at[pl.ds(base, BPW)], hidx)
    pltpu.sync_copy(tail_hbm.at[pl.ds(base, BPW)], tidx)
    pltpu.sync_copy(rel_hbm.at[pl.ds(base, BPW)], ridx)

    for c in range(BPW // C):
        off = c * C
        cph = pltpu.async_copy(ent_hbm.at[hidx.at[pl.ds(off, C)]], hrows, sem)
        cpt = pltpu.async_copy(ent_hbm.at[tidx.at[pl.ds(off, C)]], trows, sem)
        cpr = pltpu.async_copy(relt_hbm.at[ridx.at[pl.ds(off, C)]], rrows, sem)
        cpw = pltpu.async_copy(nrm_hbm.at[ridx.at[pl.ds(off, C)]], wrows, sem)
        cph.wait()
        cpt.wait()
        cpr.wait()
        cpw.wait()

        lane = lax.iota(jnp.int32, 16)

        def group(g, carry):
            acc = jnp.zeros((16,), jnp.float32)
            for j in range(16):
                e = g * 16 + j
                u0 = hrows[e, pl.ds(0, 16)] - trows[e, pl.ds(0, 16)]
                u1 = hrows[e, pl.ds(16, 16)] - trows[e, pl.ds(16, 16)]
                u2 = hrows[e, pl.ds(32, 16)] - trows[e, pl.ds(32, 16)]
                u3 = hrows[e, pl.ds(48, 16)] - trows[e, pl.ds(48, 16)]
                w0 = wrows[e, pl.ds(0, 16)]
                w1 = wrows[e, pl.ds(16, 16)]
                w2 = wrows[e, pl.ds(32, 16)]
                w3 = wrows[e, pl.ds(48, 16)]
                m = (u0 * w0 + u1 * w1) + (u2 * w2 + u3 * w3)
                a = jnp.sum(m)
                x0 = u0 + rrows[e, pl.ds(0, 16)] - a * w0
                x1 = u1 + rrows[e, pl.ds(16, 16)] - a * w1
                x2 = u2 + rrows[e, pl.ds(32, 16)] - a * w2
                x3 = u3 + rrows[e, pl.ds(48, 16)] - a * w3
                s = (jnp.abs(x0) + jnp.abs(x1)) + (jnp.abs(x2) + jnp.abs(x3))
                acc = jnp.where(lane == j, jnp.sum(s), acc)
            oscr[pl.ds(off + g * 16, 16)] = acc
            return carry

        lax.fori_loop(0, C // 16, group, None)

    pltpu.sync_copy(oscr, out_hbm.at[pl.ds(base, BPW)])


def kernel(head, relation, tail, entity_table, relation_table, normal_table):
    mesh = plsc.VectorSubcoreMesh(core_axis_name="c", subcore_axis_name="s")
    k = functools.partial(
        pl.kernel,
        mesh=mesh,
        compiler_params=pltpu.CompilerParams(
            needs_layout_passes=False, use_tc_tiling_on_sc=False),
        out_type=jax.ShapeDtypeStruct((B,), jnp.float32),
        scratch_types=[
            pltpu.VMEM((BPW,), jnp.int32),
            pltpu.VMEM((BPW,), jnp.int32),
            pltpu.VMEM((BPW,), jnp.int32),
            pltpu.VMEM((C, D), jnp.float32),
            pltpu.VMEM((C, D), jnp.float32),
            pltpu.VMEM((C, D), jnp.float32),
            pltpu.VMEM((C, D), jnp.float32),
            pltpu.VMEM((BPW,), jnp.float32),
            pltpu.SemaphoreType.DMA,
        ],
    )(_tec_body)
    return k(head, relation, tail, entity_table, relation_table,
             normal_table)
